# baseline (speedup 1.0000x reference)
import functools

import jax
import jax.numpy as jnp
from jax import lax
from jax.experimental import pallas as pl
from jax.experimental.pallas import tpu as pltpu
from jax.experimental.pallas import tpu_sc as plsc

EMB_K = 64
N_WORKERS = 32
CW = 384
NCH_FULL = 2604
CPW = 82
TAIL_C0 = NCH_FULL * CW
RING = 32
MAX_OUT = 24


def _make_gather_kernel(batch, emb_k, n_rows):
    tail_w = n_rows - TAIL_C0
    mesh = plsc.VectorSubcoreMesh(core_axis_name="c", subcore_axis_name="s")

    @functools.partial(
        pl.kernel,
        mesh=mesh,
        compiler_params=pltpu.CompilerParams(needs_layout_passes=False),
        out_type=[
            jax.ShapeDtypeStruct((batch, emb_k), jnp.float32),
            jax.ShapeDtypeStruct((batch, emb_k), jnp.float32),
        ],
        scratch_types=[
            pltpu.VMEM((batch,), jnp.int32),
            pltpu.VMEM((batch,), jnp.int32),
            pltpu.VMEM((40,), jnp.int32),
            pltpu.VMEM((3, emb_k, CW), jnp.float32),
            pltpu.VMEM((emb_k, n_rows - TAIL_C0), jnp.float32),
            pltpu.VMEM((RING, emb_k), jnp.float32),
            pltpu.SemaphoreType.DMA,
            pltpu.SemaphoreType.DMA,
        ],
    )
    def gather_kernel(usv_hbm, uov_hbm, isv_hbm, iov_hbm,
                      uwb_hbm, iwb_hbm, utabT_hbm, itabT_hbm,
                      uout_hbm, iout_hbm,
                      sval, sord, wb, bbuf, tbuf, slots, semb, semo):
        wid = lax.axis_index("s") * 2 + lax.axis_index("c")
        lane = lax.iota(jnp.int32, 16)

        def extract(ref, pos):
            base = (pos // 16) * 16
            v = ref[pl.ds(base, 16)]
            return jnp.sum(jnp.where(lane == pos - base, v, 0))

        def wait_chunk():
            pltpu.make_async_copy(
                utabT_hbm.at[:, pl.ds(0, CW)], bbuf.at[0], semb).wait()

        def wait_row():
            pltpu.make_async_copy(
                uout_hbm.at[pl.ds(0, 1)], slots.at[pl.ds(0, 1)], semo).wait()

        def do_hit(out, h, n, gather_col):
            v = extract(sval, h)
            p = extract(sord, h)
            slot = n % RING
            for q in range(4):
                col = gather_col(v, q)
                slots[slot, pl.ds(q * 16, 16)] = col
            pltpu.async_copy(slots.at[pl.ds(slot, 1)], out.at[pl.ds(p, 1)],
                             semo)
            n = n + 1

            @pl.when(n > MAX_OUT)
            def _():
                wait_row()
            return n

        def do_table(tabT, sv_hbm, so_hbm, wb_hbm, out, n):
            pltpu.sync_copy(sv_hbm, sval)
            pltpu.sync_copy(so_hbm, sord)
            pltpu.sync_copy(wb_hbm, wb)
            he_w = extract(wb, wid + 1)
            ptr0 = extract(wb, wid)

            def start(c):
                cg = jnp.minimum(wid * CPW + c, NCH_FULL - 1)
                pltpu.async_copy(tabT.at[:, pl.ds(cg * CW, CW)],
                                 bbuf.at[c % 3], semb)

            start(0)
            start(1)

            def chunk_body(c, carry):
                ptr, n = carry
                start(c + 2)
                wait_chunk()
                limit = jnp.minimum((wid * CPW + c + 1) * CW, TAIL_C0)
                par = (c % 3) + lane * 0

                def col_from_chunk(v, q):
                    kvec = q * 16 + lane
                    lvec = (v % CW) + lane * 0
                    return plsc.load_gather(bbuf, [par, kvec, lvec])

                def cond(s):
                    p_, _ = s
                    return jnp.logical_and(p_ < he_w,
                                           extract(sval, p_) < limit)

                def body(s):
                    p_, n_ = s
                    n_ = do_hit(out, p_, n_, col_from_chunk)
                    return p_ + 1, n_

                return lax.while_loop(cond, body, (ptr, n))

            ptr, n = lax.fori_loop(0, CPW, chunk_body, (ptr0, n))
            wait_chunk()
            wait_chunk()

            @pl.when(wid == N_WORKERS - 1)
            def _():
                pltpu.sync_copy(tabT.at[:, pl.ds(TAIL_C0, tail_w)], tbuf)

            def tail_col(v, q):
                kvec = q * 16 + lane
                lvec = (v - TAIL_C0) + lane * 0
                return plsc.load_gather(tbuf, [kvec, lvec])

            def tail_cond(s):
                p_, _ = s
                return p_ < he_w

            def tail_body(s):
                p_, n_ = s
                n_ = do_hit(out, p_, n_, tail_col)
                return p_ + 1, n_

            ptr, n = lax.while_loop(tail_cond, tail_body, (ptr, n))
            return n

        n = do_table(utabT_hbm, usv_hbm, uov_hbm, uwb_hbm, uout_hbm, 0)
        n = do_table(itabT_hbm, isv_hbm, iov_hbm, iwb_hbm, iout_hbm, n)

        def drain(_, c):
            wait_row()
            return c
        lax.fori_loop(0, jnp.minimum(n, MAX_OUT), drain, 0)

    return gather_kernel


def _mlp_body(u_ref, i_ref, w1_ref, b1_ref, w2_ref, out_ref):
    u = u_ref[...]
    it = i_ref[...]
    w1 = w1_ref[...]
    wa = w1[:, :EMB_K]
    wb = w1[:, EMB_K:]
    dn = (((1,), (1,)), ((), ()))
    h = lax.dot_general(u, wa, dn, preferred_element_type=jnp.float32)
    h = h + lax.dot_general(it, wb, dn, preferred_element_type=jnp.float32)
    h = jnp.maximum(h + b1_ref[...], 0.0)
    out_ref[...] = lax.dot_general(h, w2_ref[...], dn,
                                   preferred_element_type=jnp.float32)


def _mlp(u, it, W1, b1, W2, blk):
    batch = u.shape[0]
    return pl.pallas_call(
        _mlp_body,
        grid=(batch // blk,),
        in_specs=[
            pl.BlockSpec((blk, EMB_K), lambda b: (b, 0)),
            pl.BlockSpec((blk, EMB_K), lambda b: (b, 0)),
            pl.BlockSpec((EMB_K, 2 * EMB_K), lambda b: (0, 0)),
            pl.BlockSpec((1, EMB_K), lambda b: (0, 0)),
            pl.BlockSpec((1, EMB_K), lambda b: (0, 0)),
        ],
        out_specs=pl.BlockSpec((blk, 1), lambda b: (b, 0)),
        out_shape=jax.ShapeDtypeStruct((batch, 1), jnp.float32),
    )(u, it, W1, b1.reshape(1, EMB_K), W2)


def _prep(idx, batch):
    pos = jnp.arange(batch, dtype=jnp.int32)
    sval, sord = lax.sort_key_val(idx, pos)
    bounds = jnp.arange(33, dtype=jnp.int32) * (CPW * CW)
    wb = jnp.searchsorted(sval, bounds, side="left").astype(jnp.int32)
    wb = jnp.pad(wb, (0, 7))
    return sval, sord, wb


def kernel(x, user_table, item_table, W1, b1, W2):
    batch = x.shape[0]
    emb_k = user_table.shape[1]
    n_rows = user_table.shape[0]
    usv, uov, uwb = _prep(x[:, 0], batch)
    isv, iov, iwb = _prep(x[:, 1], batch)
    gk = _make_gather_kernel(batch, emb_k, n_rows)
    user_embed, item_embed = gk(usv, uov, isv, iov, uwb, iwb,
                                user_table.T, item_table.T)
    out = _mlp(user_embed, item_embed, W1, b1, W2, blk=2048)
    return (out, user_embed, item_embed)
